# baseline (speedup 1.0000x reference)
import functools

import jax
import jax.numpy as jnp
from jax import lax
from jax.experimental import pallas as pl
from jax.experimental.pallas import tpu as pltpu
from jax.experimental.pallas import tpu_sc as plsc

N = 10000
E = 320000
D_IN = 128
D_HID = 64

NC = 2
NS = 16
NW = NC * NS
EPW = E // NW
BLK = 125
NBLK = EPW // BLK
GBLK = 500
NG = EPW // GBLK
QPG = GBLK // BLK
N_PAD = 10240
RPS = N_PAD // NS

_sc_mesh = plsc.VectorSubcoreMesh(core_axis_name="c", subcore_axis_name="s")
_sc_params = pltpu.CompilerParams(use_tc_tiling_on_sc=False)


@functools.partial(
    pl.kernel,
    mesh=_sc_mesh,
    out_type=jax.ShapeDtypeStruct((N_PAD, 128), jnp.float32),
    scratch_types=[
        pltpu.VMEM((NBLK, BLK), jnp.int32),
        pltpu.VMEM((BLK, 16), jnp.float32),
        pltpu.VMEM_SHARED((N_PAD, 16), jnp.float32),
    ],
    compiler_params=_sc_params,
)
def _deg_sc(dst_hbm, zeros_hbm, ones_hbm, out_hbm, dst_v, ones_v, acc):
    c = lax.axis_index("c")
    s = lax.axis_index("s")
    wid = c * NS + s
    rows = pl.ds(s * RPS, RPS)
    pltpu.sync_copy(zeros_hbm.at[rows], acc.at[rows])
    pltpu.sync_copy(ones_hbm, ones_v)
    pltpu.sync_copy(dst_hbm.at[wid], dst_v)
    plsc.subcore_barrier()

    @pl.loop(0, NBLK)
    def _(j):
        pltpu.sync_copy(ones_v, acc.at[dst_v.at[j]], add=True)

    plsc.subcore_barrier()
    pltpu.sync_copy(acc.at[rows], out_hbm.at[rows, pl.ds(c * 16, 16)])


@functools.partial(
    pl.kernel,
    mesh=_sc_mesh,
    out_type=jax.ShapeDtypeStruct((N_PAD, NC * D_HID), jnp.float32),
    scratch_types=[
        pltpu.VMEM((NG, GBLK), jnp.int32),
        pltpu.VMEM((NBLK, BLK), jnp.int32),
        pltpu.VMEM((GBLK, D_HID), jnp.float32),
        pltpu.VMEM((GBLK, D_HID), jnp.float32),
        pltpu.VMEM_SHARED((N_PAD, D_HID), jnp.float32),
        pltpu.SemaphoreType.DMA,
        pltpu.SemaphoreType.DMA,
        pltpu.SemaphoreType.DMA,
        pltpu.SemaphoreType.DMA,
    ],
    compiler_params=_sc_params,
)
def _agg_sc(h_hbm, src_hbm, dst_hbm, zeros_hbm, out_hbm,
            src_v, dst_v, buf0, buf1, acc, gsem0, gsem1, ssem0, ssem1):
    c = lax.axis_index("c")
    s = lax.axis_index("s")
    wid = c * NS + s
    rows = pl.ds(s * RPS, RPS)
    pltpu.sync_copy(zeros_hbm.at[rows], acc.at[rows])
    pltpu.sync_copy(src_hbm.at[wid], src_v)
    pltpu.sync_copy(dst_hbm.at[wid], dst_v)
    plsc.subcore_barrier()

    pltpu.async_copy(h_hbm.at[src_v.at[0]], buf0, gsem0)
    pltpu.async_copy(h_hbm.at[src_v.at[1]], buf1, gsem1)

    @pl.loop(0, NG // 2)
    def _(i):
        m0 = 2 * i

        for m, buf, sem, ssem in ((m0, buf0, gsem0, ssem0),
                                  (m0 + 1, buf1, gsem1, ssem1)):
            pltpu.make_async_copy(h_hbm.at[src_v.at[m]], buf, sem).wait()
            for q in range(QPG):
                pltpu.async_copy(buf.at[pl.ds(q * BLK, BLK)],
                                 acc.at[dst_v.at[m * QPG + q]], ssem, add=True)
            for q in range(QPG):
                pltpu.make_async_copy(buf.at[pl.ds(q * BLK, BLK)],
                                      acc.at[dst_v.at[m * QPG + q]], ssem).wait()
            pltpu.async_copy(h_hbm.at[src_v.at[lax.rem(m + 2, NG)]], buf, sem)

    pltpu.make_async_copy(h_hbm.at[src_v.at[0]], buf0, gsem0).wait()
    pltpu.make_async_copy(h_hbm.at[src_v.at[1]], buf1, gsem1).wait()
    plsc.subcore_barrier()
    pltpu.sync_copy(acc.at[rows], out_hbm.at[rows, pl.ds(c * D_HID, D_HID)])


TB = 2000
_TGRID = N // TB


def _head_body(dacc_ref, x_ref, w1_ref, dis_ref, hp_ref):
    deg = dacc_ref[:, 0:1] + dacc_ref[:, 16:17] + 1.0
    dis = lax.rsqrt(deg)
    dis_ref[...] = dis
    h = jnp.dot(x_ref[...], w1_ref[...], preferred_element_type=jnp.float32)
    hp_ref[...] = h * dis


_head = pl.pallas_call(
    _head_body,
    grid=(_TGRID,),
    in_specs=[
        pl.BlockSpec((TB, 128), lambda i: (i, 0)),
        pl.BlockSpec((TB, D_IN), lambda i: (i, 0)),
        pl.BlockSpec((D_IN, D_HID), lambda i: (0, 0)),
    ],
    out_specs=(pl.BlockSpec((TB, 1), lambda i: (i, 0)),
               pl.BlockSpec((TB, D_HID), lambda i: (i, 0))),
    out_shape=(jax.ShapeDtypeStruct((N, 1), jnp.float32),
               jax.ShapeDtypeStruct((N, D_HID), jnp.float32)))


def _mid_body(p_ref, hp_ref, dis_ref, b1_ref, w2_ref, o_ref):
    dis = dis_ref[...]
    p = p_ref[:, :D_HID] + p_ref[:, D_HID:]
    z = dis * (p + hp_ref[...]) + b1_ref[...]
    z = jnp.maximum(z, 0.0)
    o_ref[...] = jnp.dot(z, w2_ref[...], preferred_element_type=jnp.float32) * dis


_mid = pl.pallas_call(
    _mid_body,
    grid=(_TGRID,),
    in_specs=[
        pl.BlockSpec((TB, NC * D_HID), lambda i: (i, 0)),
        pl.BlockSpec((TB, D_HID), lambda i: (i, 0)),
        pl.BlockSpec((TB, 1), lambda i: (i, 0)),
        pl.BlockSpec((1, D_HID), lambda i: (0, 0)),
        pl.BlockSpec((D_HID, D_HID), lambda i: (0, 0)),
    ],
    out_specs=pl.BlockSpec((TB, D_HID), lambda i: (i, 0)),
    out_shape=jax.ShapeDtypeStruct((N, D_HID), jnp.float32))


def _fin_body(p_ref, hp_ref, dis_ref, b2_ref, o_ref):
    p = p_ref[:, :D_HID] + p_ref[:, D_HID:]
    o_ref[...] = (dis_ref[...] * (p + hp_ref[...]) + b2_ref[...])


_fin = pl.pallas_call(
    _fin_body,
    grid=(_TGRID,),
    in_specs=[
        pl.BlockSpec((TB, NC * D_HID), lambda i: (i, 0)),
        pl.BlockSpec((TB, D_HID), lambda i: (i, 0)),
        pl.BlockSpec((TB, 1), lambda i: (i, 0)),
        pl.BlockSpec((1, D_HID), lambda i: (0, 0)),
    ],
    out_specs=pl.BlockSpec((TB, D_HID), lambda i: (i, 0)),
    out_shape=jax.ShapeDtypeStruct((N, D_HID), jnp.float32))


def kernel(x, edge_index, W1, b1, W2, b2):
    src = edge_index[0].reshape(NW, NG, GBLK)
    dst = edge_index[1].reshape(NW, NBLK, BLK)
    z64 = jnp.zeros((N_PAD, D_HID), jnp.float32)
    z16 = jnp.zeros((N_PAD, 16), jnp.float32)
    on16 = jnp.ones((BLK, 16), jnp.float32)
    b1r = b1.reshape(1, D_HID)
    b2r = b2.reshape(1, D_HID)

    dacc = _deg_sc(dst, z16, on16)
    dis, h1p = _head(dacc, x, W1)
    p1 = _agg_sc(h1p, src, dst, z64)
    h2p = _mid(p1, h1p, dis, b1r, W2)
    p2 = _agg_sc(h2p, src, dst, z64)
    return _fin(p2, h2p, dis, b2r)


# speedup vs baseline: 1.0025x; 1.0025x over previous
import functools

import jax
import jax.numpy as jnp
from jax import lax
from jax.experimental import pallas as pl
from jax.experimental.pallas import tpu as pltpu
from jax.experimental.pallas import tpu_sc as plsc

N = 10000
E = 320000
D_IN = 128
D_HID = 64

NC = 2
NS = 16
NW = NC * NS
EPW = E // NW
BLK = 125
NBLK = EPW // BLK
GBLK = 500
NG = EPW // GBLK
QPG = GBLK // BLK
N_PAD = 10240
RPS = N_PAD // NS

_sc_mesh = plsc.VectorSubcoreMesh(core_axis_name="c", subcore_axis_name="s")
_sc_params = pltpu.CompilerParams(use_tc_tiling_on_sc=False)


@functools.partial(
    pl.kernel,
    mesh=_sc_mesh,
    out_type=jax.ShapeDtypeStruct((N_PAD, 128), jnp.float32),
    scratch_types=[
        pltpu.VMEM((NBLK, BLK), jnp.int32),
        pltpu.VMEM((BLK, 16), jnp.float32),
        pltpu.VMEM_SHARED((N_PAD, 16), jnp.float32),
    ],
    compiler_params=_sc_params,
)
def _deg_sc(dst_hbm, zeros_hbm, ones_hbm, out_hbm, dst_v, ones_v, acc):
    c = lax.axis_index("c")
    s = lax.axis_index("s")
    wid = c * NS + s
    rows = pl.ds(s * RPS, RPS)
    pltpu.sync_copy(zeros_hbm.at[rows], acc.at[rows])
    pltpu.sync_copy(ones_hbm, ones_v)
    pltpu.sync_copy(dst_hbm.at[wid], dst_v)
    plsc.subcore_barrier()

    @pl.loop(0, NBLK)
    def _(j):
        pltpu.sync_copy(ones_v, acc.at[dst_v.at[j]], add=True)

    plsc.subcore_barrier()
    pltpu.sync_copy(acc.at[rows], out_hbm.at[rows, pl.ds(c * 16, 16)])


@functools.partial(
    pl.kernel,
    mesh=_sc_mesh,
    out_type=jax.ShapeDtypeStruct((N_PAD, NC * D_HID), jnp.float32),
    scratch_types=[
        pltpu.VMEM((NG, GBLK), jnp.int32),
        pltpu.VMEM((NBLK, BLK), jnp.int32),
        pltpu.VMEM((GBLK, D_HID), jnp.float32),
        pltpu.VMEM((GBLK, D_HID), jnp.float32),
        pltpu.VMEM_SHARED((N_PAD, D_HID), jnp.float32),
        pltpu.SemaphoreType.DMA,
        pltpu.SemaphoreType.DMA,
    ],
    compiler_params=_sc_params,
)
def _agg_sc(h_hbm, src_hbm, dst_hbm, zeros_hbm, out_hbm,
            src_v, dst_v, buf0, buf1, acc, gsem0, gsem1):
    c = lax.axis_index("c")
    s = lax.axis_index("s")
    wid = c * NS + s
    rows = pl.ds(s * RPS, RPS)
    pltpu.sync_copy(zeros_hbm.at[rows], acc.at[rows])
    pltpu.sync_copy(src_hbm.at[wid], src_v)
    pltpu.sync_copy(dst_hbm.at[wid], dst_v)
    plsc.subcore_barrier()

    pltpu.async_copy(h_hbm.at[src_v.at[0]], buf0, gsem0)
    pltpu.async_copy(h_hbm.at[src_v.at[1]], buf1, gsem1)

    @pl.loop(0, NG // 2)
    def _(i):
        m0 = 2 * i

        for m, buf, sem in ((m0, buf0, gsem0), (m0 + 1, buf1, gsem1)):
            pltpu.make_async_copy(h_hbm.at[src_v.at[m]], buf, sem).wait()
            for q in range(QPG):
                pltpu.sync_copy(buf.at[pl.ds(q * BLK, BLK)],
                                acc.at[dst_v.at[m * QPG + q]], add=True)
            pltpu.async_copy(h_hbm.at[src_v.at[lax.rem(m + 2, NG)]], buf, sem)

    pltpu.make_async_copy(h_hbm.at[src_v.at[0]], buf0, gsem0).wait()
    pltpu.make_async_copy(h_hbm.at[src_v.at[1]], buf1, gsem1).wait()
    plsc.subcore_barrier()
    pltpu.sync_copy(acc.at[rows], out_hbm.at[rows, pl.ds(c * D_HID, D_HID)])


TB = 2000
_TGRID = N // TB


def _head_body(dacc_ref, x_ref, w1_ref, dis_ref, hp_ref):
    deg = dacc_ref[:, 0:1] + dacc_ref[:, 16:17] + 1.0
    dis = lax.rsqrt(deg)
    dis_ref[...] = dis
    h = jnp.dot(x_ref[...], w1_ref[...], preferred_element_type=jnp.float32)
    hp_ref[...] = h * dis


_head = pl.pallas_call(
    _head_body,
    grid=(_TGRID,),
    in_specs=[
        pl.BlockSpec((TB, 128), lambda i: (i, 0)),
        pl.BlockSpec((TB, D_IN), lambda i: (i, 0)),
        pl.BlockSpec((D_IN, D_HID), lambda i: (0, 0)),
    ],
    out_specs=(pl.BlockSpec((TB, 1), lambda i: (i, 0)),
               pl.BlockSpec((TB, D_HID), lambda i: (i, 0))),
    out_shape=(jax.ShapeDtypeStruct((N, 1), jnp.float32),
               jax.ShapeDtypeStruct((N, D_HID), jnp.float32)))


def _mid_body(p_ref, hp_ref, dis_ref, b1_ref, w2_ref, o_ref):
    dis = dis_ref[...]
    p = p_ref[:, :D_HID] + p_ref[:, D_HID:]
    z = dis * (p + hp_ref[...]) + b1_ref[...]
    z = jnp.maximum(z, 0.0)
    o_ref[...] = jnp.dot(z, w2_ref[...], preferred_element_type=jnp.float32) * dis


_mid = pl.pallas_call(
    _mid_body,
    grid=(_TGRID,),
    in_specs=[
        pl.BlockSpec((TB, NC * D_HID), lambda i: (i, 0)),
        pl.BlockSpec((TB, D_HID), lambda i: (i, 0)),
        pl.BlockSpec((TB, 1), lambda i: (i, 0)),
        pl.BlockSpec((1, D_HID), lambda i: (0, 0)),
        pl.BlockSpec((D_HID, D_HID), lambda i: (0, 0)),
    ],
    out_specs=pl.BlockSpec((TB, D_HID), lambda i: (i, 0)),
    out_shape=jax.ShapeDtypeStruct((N, D_HID), jnp.float32))


def _fin_body(p_ref, hp_ref, dis_ref, b2_ref, o_ref):
    p = p_ref[:, :D_HID] + p_ref[:, D_HID:]
    o_ref[...] = (dis_ref[...] * (p + hp_ref[...]) + b2_ref[...])


_fin = pl.pallas_call(
    _fin_body,
    grid=(_TGRID,),
    in_specs=[
        pl.BlockSpec((TB, NC * D_HID), lambda i: (i, 0)),
        pl.BlockSpec((TB, D_HID), lambda i: (i, 0)),
        pl.BlockSpec((TB, 1), lambda i: (i, 0)),
        pl.BlockSpec((1, D_HID), lambda i: (0, 0)),
    ],
    out_specs=pl.BlockSpec((TB, D_HID), lambda i: (i, 0)),
    out_shape=jax.ShapeDtypeStruct((N, D_HID), jnp.float32))


def kernel(x, edge_index, W1, b1, W2, b2):
    src = edge_index[0].reshape(NW, NG, GBLK)
    dst = edge_index[1].reshape(NW, NBLK, BLK)
    z64 = jnp.zeros((N_PAD, D_HID), jnp.float32)
    z16 = jnp.zeros((N_PAD, 16), jnp.float32)
    on16 = jnp.ones((BLK, 16), jnp.float32)
    b1r = b1.reshape(1, D_HID)
    b2r = b2.reshape(1, D_HID)

    dacc = _deg_sc(dst, z16, on16)
    dis, h1p = _head(dacc, x, W1)
    p1 = _agg_sc(h1p, src, dst, z64)
    h2p = _mid(p1, h1p, dis, b1r, W2)
    p2 = _agg_sc(h2p, src, dst, z64)
    return _fin(p2, h2p, dis, b2r)


# speedup vs baseline: 1.0224x; 1.0198x over previous
import functools

import jax
import jax.numpy as jnp
from jax import lax
from jax.experimental import pallas as pl
from jax.experimental.pallas import tpu as pltpu
from jax.experimental.pallas import tpu_sc as plsc

N = 10000
E = 320000
D_IN = 128
D_HID = 64

NC = 2
NS = 16
NW = NC * NS
EPW = E // NW
BLK = 125
NBLK = EPW // BLK
GBLK = 500
NG = EPW // GBLK
QPG = GBLK // BLK
N_PAD = 10240
RPS = N_PAD // NS

_sc_mesh = plsc.VectorSubcoreMesh(core_axis_name="c", subcore_axis_name="s")
_sc_params = pltpu.CompilerParams(use_tc_tiling_on_sc=False)


@functools.partial(
    pl.kernel,
    mesh=_sc_mesh,
    out_type=jax.ShapeDtypeStruct((N_PAD, 128), jnp.float32),
    scratch_types=[
        pltpu.VMEM((NBLK, BLK), jnp.int32),
        pltpu.VMEM((BLK, 16), jnp.float32),
        pltpu.VMEM_SHARED((N_PAD, 16), jnp.float32),
    ],
    compiler_params=_sc_params,
)
def _deg_sc(dst_hbm, zeros_hbm, ones_hbm, out_hbm, dst_v, ones_v, acc):
    c = lax.axis_index("c")
    s = lax.axis_index("s")
    wid = c * NS + s
    rows = pl.ds(s * RPS, RPS)
    pltpu.sync_copy(zeros_hbm.at[rows], acc.at[rows])
    pltpu.sync_copy(ones_hbm, ones_v)
    pltpu.sync_copy(dst_hbm.at[wid], dst_v)
    plsc.subcore_barrier()

    @pl.loop(0, NBLK)
    def _(j):
        pltpu.sync_copy(ones_v, acc.at[dst_v.at[j]], add=True)

    plsc.subcore_barrier()
    pltpu.sync_copy(acc.at[rows], out_hbm.at[rows, pl.ds(c * 16, 16)])


@functools.partial(
    pl.kernel,
    mesh=_sc_mesh,
    out_type=jax.ShapeDtypeStruct((N_PAD, NC * D_HID), jnp.float32),
    scratch_types=[
        pltpu.VMEM((NG, GBLK), jnp.int32),
        pltpu.VMEM((NBLK, BLK), jnp.int32),
        pltpu.VMEM((GBLK, D_HID), jnp.float32),
        pltpu.VMEM((GBLK, D_HID), jnp.float32),
        pltpu.VMEM_SHARED((N_PAD, D_HID), jnp.float32),
        pltpu.SemaphoreType.DMA,
        pltpu.SemaphoreType.DMA,
    ],
    compiler_params=_sc_params,
)
def _agg_sc(h_hbm, src_hbm, dst_hbm, zeros_hbm, out_hbm,
            src_v, dst_v, buf0, buf1, acc, gsem0, gsem1):
    c = lax.axis_index("c")
    s = lax.axis_index("s")
    wid = c * NS + s
    rows = pl.ds(s * RPS, RPS)
    pltpu.async_copy(zeros_hbm.at[rows], acc.at[rows], gsem0)
    pltpu.sync_copy(src_hbm.at[wid], src_v)
    pltpu.sync_copy(dst_hbm.at[wid], dst_v)
    pltpu.make_async_copy(zeros_hbm.at[rows], acc.at[rows], gsem0).wait()
    plsc.subcore_barrier()

    pltpu.async_copy(h_hbm.at[src_v.at[0]], buf0, gsem0)
    pltpu.async_copy(h_hbm.at[src_v.at[1]], buf1, gsem1)

    @pl.loop(0, NG // 2)
    def _(i):
        m0 = 2 * i

        for m, buf, sem in ((m0, buf0, gsem0), (m0 + 1, buf1, gsem1)):
            pltpu.make_async_copy(h_hbm.at[src_v.at[m]], buf, sem).wait()
            for q in range(QPG):
                pltpu.sync_copy(buf.at[pl.ds(q * BLK, BLK)],
                                acc.at[dst_v.at[m * QPG + q]], add=True)
            pltpu.async_copy(h_hbm.at[src_v.at[lax.rem(m + 2, NG)]], buf, sem)

    pltpu.make_async_copy(h_hbm.at[src_v.at[0]], buf0, gsem0).wait()
    pltpu.make_async_copy(h_hbm.at[src_v.at[1]], buf1, gsem1).wait()
    plsc.subcore_barrier()
    pltpu.sync_copy(acc.at[rows], out_hbm.at[rows, pl.ds(c * D_HID, D_HID)])


TB = 2000
_TGRID = N // TB


def _head_body(dacc_ref, x_ref, w1_ref, dis_ref, hp_ref):
    deg = dacc_ref[:, 0:1] + dacc_ref[:, 16:17] + 1.0
    dis = lax.rsqrt(deg)
    dis_ref[...] = dis
    h = jnp.dot(x_ref[...], w1_ref[...], preferred_element_type=jnp.float32)
    hp_ref[...] = h * dis


_head = pl.pallas_call(
    _head_body,
    grid=(_TGRID,),
    in_specs=[
        pl.BlockSpec((TB, 128), lambda i: (i, 0)),
        pl.BlockSpec((TB, D_IN), lambda i: (i, 0)),
        pl.BlockSpec((D_IN, D_HID), lambda i: (0, 0)),
    ],
    out_specs=(pl.BlockSpec((TB, 1), lambda i: (i, 0)),
               pl.BlockSpec((TB, D_HID), lambda i: (i, 0))),
    out_shape=(jax.ShapeDtypeStruct((N, 1), jnp.float32),
               jax.ShapeDtypeStruct((N, D_HID), jnp.float32)))


def _mid_body(p_ref, hp_ref, dis_ref, b1_ref, w2_ref, o_ref):
    dis = dis_ref[...]
    p = p_ref[:, :D_HID] + p_ref[:, D_HID:]
    z = dis * (p + hp_ref[...]) + b1_ref[...]
    z = jnp.maximum(z, 0.0)
    o_ref[...] = jnp.dot(z, w2_ref[...], preferred_element_type=jnp.float32) * dis


_mid = pl.pallas_call(
    _mid_body,
    grid=(_TGRID,),
    in_specs=[
        pl.BlockSpec((TB, NC * D_HID), lambda i: (i, 0)),
        pl.BlockSpec((TB, D_HID), lambda i: (i, 0)),
        pl.BlockSpec((TB, 1), lambda i: (i, 0)),
        pl.BlockSpec((1, D_HID), lambda i: (0, 0)),
        pl.BlockSpec((D_HID, D_HID), lambda i: (0, 0)),
    ],
    out_specs=pl.BlockSpec((TB, D_HID), lambda i: (i, 0)),
    out_shape=jax.ShapeDtypeStruct((N, D_HID), jnp.float32))


def _fin_body(p_ref, hp_ref, dis_ref, b2_ref, o_ref):
    p = p_ref[:, :D_HID] + p_ref[:, D_HID:]
    o_ref[...] = (dis_ref[...] * (p + hp_ref[...]) + b2_ref[...])


_fin = pl.pallas_call(
    _fin_body,
    grid=(_TGRID,),
    in_specs=[
        pl.BlockSpec((TB, NC * D_HID), lambda i: (i, 0)),
        pl.BlockSpec((TB, D_HID), lambda i: (i, 0)),
        pl.BlockSpec((TB, 1), lambda i: (i, 0)),
        pl.BlockSpec((1, D_HID), lambda i: (0, 0)),
    ],
    out_specs=pl.BlockSpec((TB, D_HID), lambda i: (i, 0)),
    out_shape=jax.ShapeDtypeStruct((N, D_HID), jnp.float32))


def kernel(x, edge_index, W1, b1, W2, b2):
    src = edge_index[0].reshape(NW, NG, GBLK)
    dst = edge_index[1].reshape(NW, NBLK, BLK)
    z64 = jnp.zeros((N_PAD, D_HID), jnp.float32)
    z16 = jnp.zeros((N_PAD, 16), jnp.float32)
    on16 = jnp.ones((BLK, 16), jnp.float32)
    b1r = b1.reshape(1, D_HID)
    b2r = b2.reshape(1, D_HID)

    dacc = _deg_sc(dst, z16, on16)
    dis, h1p = _head(dacc, x, W1)
    p1 = _agg_sc(h1p, src, dst, z64)
    h2p = _mid(p1, h1p, dis, b1r, W2)
    p2 = _agg_sc(h2p, src, dst, z64)
    return _fin(p2, h2p, dis, b2r)


# speedup vs baseline: 1.0249x; 1.0025x over previous
import functools

import jax
import jax.numpy as jnp
from jax import lax
from jax.experimental import pallas as pl
from jax.experimental.pallas import tpu as pltpu
from jax.experimental.pallas import tpu_sc as plsc

N = 10000
E = 320000
D_IN = 128
D_HID = 64

NC = 2
NS = 16
NW = NC * NS
EPW = E // NW
BLK = 125
NBLK = EPW // BLK
GBLK = 500
NG = EPW // GBLK
QPG = GBLK // BLK
N_PAD = 10240
RPS = N_PAD // NS

_sc_mesh = plsc.VectorSubcoreMesh(core_axis_name="c", subcore_axis_name="s")
_sc_params = pltpu.CompilerParams(use_tc_tiling_on_sc=False)


@functools.partial(
    pl.kernel,
    mesh=_sc_mesh,
    out_type=jax.ShapeDtypeStruct((N_PAD, 128), jnp.float32),
    scratch_types=[
        pltpu.VMEM((NBLK, BLK), jnp.int32),
        pltpu.VMEM((BLK, 16), jnp.float32),
        pltpu.VMEM_SHARED((N_PAD, 16), jnp.float32),
        pltpu.SemaphoreType.DMA,
    ],
    compiler_params=_sc_params,
)
def _deg_sc(dst_hbm, zeros_hbm, ones_hbm, out_hbm, dst_v, ones_v, acc, sem):
    c = lax.axis_index("c")
    s = lax.axis_index("s")
    wid = c * NS + s
    rows = pl.ds(s * RPS, RPS)
    pltpu.async_copy(zeros_hbm.at[rows], acc.at[rows], sem)
    pltpu.sync_copy(ones_hbm, ones_v)
    pltpu.sync_copy(dst_hbm.at[wid], dst_v)
    pltpu.make_async_copy(zeros_hbm.at[rows], acc.at[rows], sem).wait()
    plsc.subcore_barrier()

    @pl.loop(0, NBLK)
    def _(j):
        pltpu.sync_copy(ones_v, acc.at[dst_v.at[j]], add=True)

    plsc.subcore_barrier()
    pltpu.sync_copy(acc.at[rows], out_hbm.at[rows, pl.ds(c * 16, 16)])


@functools.partial(
    pl.kernel,
    mesh=_sc_mesh,
    out_type=jax.ShapeDtypeStruct((N_PAD, NC * D_HID), jnp.float32),
    scratch_types=[
        pltpu.VMEM((NG, GBLK), jnp.int32),
        pltpu.VMEM((NBLK, BLK), jnp.int32),
        pltpu.VMEM((GBLK, D_HID), jnp.float32),
        pltpu.VMEM((GBLK, D_HID), jnp.float32),
        pltpu.VMEM_SHARED((N_PAD, D_HID), jnp.float32),
        pltpu.SemaphoreType.DMA,
        pltpu.SemaphoreType.DMA,
    ],
    compiler_params=_sc_params,
)
def _agg_sc(h_hbm, src_hbm, dst_hbm, zeros_hbm, out_hbm,
            src_v, dst_v, buf0, buf1, acc, gsem0, gsem1):
    c = lax.axis_index("c")
    s = lax.axis_index("s")
    wid = c * NS + s
    rows = pl.ds(s * RPS, RPS)
    pltpu.async_copy(zeros_hbm.at[rows], acc.at[rows], gsem0)
    pltpu.sync_copy(src_hbm.at[wid], src_v)
    pltpu.sync_copy(dst_hbm.at[wid], dst_v)
    pltpu.make_async_copy(zeros_hbm.at[rows], acc.at[rows], gsem0).wait()
    plsc.subcore_barrier()

    pltpu.async_copy(h_hbm.at[src_v.at[0]], buf0, gsem0)
    pltpu.async_copy(h_hbm.at[src_v.at[1]], buf1, gsem1)

    @pl.loop(0, NG // 2)
    def _(i):
        m0 = 2 * i

        for m, buf, sem in ((m0, buf0, gsem0), (m0 + 1, buf1, gsem1)):
            pltpu.make_async_copy(h_hbm.at[src_v.at[m]], buf, sem).wait()
            for q in range(QPG):
                pltpu.sync_copy(buf.at[pl.ds(q * BLK, BLK)],
                                acc.at[dst_v.at[m * QPG + q]], add=True)
            pltpu.async_copy(h_hbm.at[src_v.at[lax.rem(m + 2, NG)]], buf, sem)

    pltpu.make_async_copy(h_hbm.at[src_v.at[0]], buf0, gsem0).wait()
    pltpu.make_async_copy(h_hbm.at[src_v.at[1]], buf1, gsem1).wait()
    plsc.subcore_barrier()
    pltpu.sync_copy(acc.at[rows], out_hbm.at[rows, pl.ds(c * D_HID, D_HID)])


TB = 2000
_TGRID = N // TB


def _head_body(dacc_ref, x_ref, w1_ref, dis_ref, hp_ref):
    deg = dacc_ref[:, 0:1] + dacc_ref[:, 16:17] + 1.0
    dis = lax.rsqrt(deg)
    dis_ref[...] = dis
    h = jnp.dot(x_ref[...], w1_ref[...], preferred_element_type=jnp.float32)
    hp_ref[...] = h * dis


_head = pl.pallas_call(
    _head_body,
    grid=(_TGRID,),
    in_specs=[
        pl.BlockSpec((TB, 128), lambda i: (i, 0)),
        pl.BlockSpec((TB, D_IN), lambda i: (i, 0)),
        pl.BlockSpec((D_IN, D_HID), lambda i: (0, 0)),
    ],
    out_specs=(pl.BlockSpec((TB, 1), lambda i: (i, 0)),
               pl.BlockSpec((TB, D_HID), lambda i: (i, 0))),
    out_shape=(jax.ShapeDtypeStruct((N, 1), jnp.float32),
               jax.ShapeDtypeStruct((N, D_HID), jnp.float32)))


def _mid_body(p_ref, hp_ref, dis_ref, b1_ref, w2_ref, o_ref):
    dis = dis_ref[...]
    p = p_ref[:, :D_HID] + p_ref[:, D_HID:]
    z = dis * (p + hp_ref[...]) + b1_ref[...]
    z = jnp.maximum(z, 0.0)
    o_ref[...] = jnp.dot(z, w2_ref[...], preferred_element_type=jnp.float32) * dis


_mid = pl.pallas_call(
    _mid_body,
    grid=(_TGRID,),
    in_specs=[
        pl.BlockSpec((TB, NC * D_HID), lambda i: (i, 0)),
        pl.BlockSpec((TB, D_HID), lambda i: (i, 0)),
        pl.BlockSpec((TB, 1), lambda i: (i, 0)),
        pl.BlockSpec((1, D_HID), lambda i: (0, 0)),
        pl.BlockSpec((D_HID, D_HID), lambda i: (0, 0)),
    ],
    out_specs=pl.BlockSpec((TB, D_HID), lambda i: (i, 0)),
    out_shape=jax.ShapeDtypeStruct((N, D_HID), jnp.float32))


def _fin_body(p_ref, hp_ref, dis_ref, b2_ref, o_ref):
    p = p_ref[:, :D_HID] + p_ref[:, D_HID:]
    o_ref[...] = (dis_ref[...] * (p + hp_ref[...]) + b2_ref[...])


_fin = pl.pallas_call(
    _fin_body,
    grid=(_TGRID,),
    in_specs=[
        pl.BlockSpec((TB, NC * D_HID), lambda i: (i, 0)),
        pl.BlockSpec((TB, D_HID), lambda i: (i, 0)),
        pl.BlockSpec((TB, 1), lambda i: (i, 0)),
        pl.BlockSpec((1, D_HID), lambda i: (0, 0)),
    ],
    out_specs=pl.BlockSpec((TB, D_HID), lambda i: (i, 0)),
    out_shape=jax.ShapeDtypeStruct((N, D_HID), jnp.float32))


def kernel(x, edge_index, W1, b1, W2, b2):
    src = edge_index[0].reshape(NW, NG, GBLK)
    dst = edge_index[1].reshape(NW, NBLK, BLK)
    z64 = jnp.zeros((N_PAD, D_HID), jnp.float32)
    z16 = jnp.zeros((N_PAD, 16), jnp.float32)
    on16 = jnp.ones((BLK, 16), jnp.float32)
    b1r = b1.reshape(1, D_HID)
    b2r = b2.reshape(1, D_HID)

    dacc = _deg_sc(dst, z16, on16)
    dis, h1p = _head(dacc, x, W1)
    p1 = _agg_sc(h1p, src, dst, z64)
    h2p = _mid(p1, h1p, dis, b1r, W2)
    p2 = _agg_sc(h2p, src, dst, z64)
    return _fin(p2, h2p, dis, b2r)
